# Initial kernel scaffold; baseline (speedup 1.0000x reference)
import jax
import jax.numpy as jnp
from jax.experimental import pallas as pl


def kernel(input_, weight_stacked, indices):
    raise NotImplementedError("write your pallas kernel here")


# speedup vs baseline: 1.3328x; 1.3328x over previous
import jax
import jax.numpy as jnp
from jax.experimental import pallas as pl

T = 2048
IN_F = 1024
OUT_F = 1024
E = 8


def _dense_body(idx_ref, x_ref, w_ref, o_ref):
    e = pl.program_id(0)

    @pl.when(e == 0)
    def _():
        o_ref[...] = jnp.zeros_like(o_ref)

    x = x_ref[...]
    w = w_ref[0]
    y = jax.lax.dot_general(
        x, w, (((1,), (1,)), ((), ())), preferred_element_type=jnp.float32
    )
    mask = idx_ref[...] == e
    o_ref[...] += jnp.where(mask, y, 0.0)


def kernel(input_, weight_stacked, indices):
    idx = indices.astype(jnp.int32).reshape(T, 1)
    x = input_.astype(jnp.bfloat16)
    w = weight_stacked.astype(jnp.bfloat16)
    out = pl.pallas_call(
        _dense_body,
        grid=(E,),
        in_specs=[
            pl.BlockSpec((T, 1), lambda e: (0, 0)),
            pl.BlockSpec((T, IN_F), lambda e: (0, 0)),
            pl.BlockSpec((1, OUT_F, IN_F), lambda e: (e, 0, 0)),
        ],
        out_specs=pl.BlockSpec((T, OUT_F), lambda e: (0, 0)),
        out_shape=jax.ShapeDtypeStruct((T, OUT_F), jnp.float32),
    )(idx, x, w)
    return out
